# Initial kernel scaffold; baseline (speedup 1.0000x reference)
import jax
import jax.numpy as jnp
from jax.experimental import pallas as pl


def kernel(node_features, edge_features, edge_index, params):
    raise NotImplementedError("write your pallas kernel here")


# speedup vs baseline: 2.2050x; 2.2050x over previous
import functools

import jax
import jax.numpy as jnp
import numpy as np
from jax import lax
from jax.experimental import pallas as pl
from jax.experimental.pallas import tpu as pltpu
from jax.experimental.pallas import tpu_sc as plsc

N_NODES = 10000
N_EDGES = 320000
HID = 128
PROC_SIZE = 15

NC, NS = 2, 16
NW = NC * NS
EW = N_EDGES // NW
R = 80
C = EW // R
N_PAD = 10240
STRIP = N_PAD // NS
HW = HID // 2

EB = 2000
NB = 1000

_mesh = plsc.VectorSubcoreMesh(core_axis_name="c", subcore_axis_name="s")


def _gather_body(pt_hbm, src_hbm, dst_hbm, out_hbm,
                 sidx, didx, bufa, bufb, obuf, sg0, sg1, so0, so1):
    cid = lax.axis_index("c")
    sid = lax.axis_index("s")
    wid = sid * NC + cid
    pltpu.sync_copy(src_hbm.at[wid], sidx)
    pltpu.sync_copy(dst_hbm.at[wid], didx)
    base = wid * EW
    sg = (sg0, sg1)
    so = (so0, so1)

    def issue(j, b):
        pltpu.async_copy(pt_hbm.at[sidx.at[j]], bufa.at[b], sg[b])
        pltpu.async_copy(pt_hbm.at[didx.at[j]], bufb.at[b], sg[b])

    def wait_gather(j, b):
        pltpu.make_async_copy(pt_hbm.at[sidx.at[j]], bufa.at[b], sg[b]).wait()
        pltpu.make_async_copy(pt_hbm.at[didx.at[j]], bufb.at[b], sg[b]).wait()

    def out_desc(j, b):
        return pltpu.make_async_copy(
            obuf.at[b], out_hbm.at[pl.ds(base + j * R, R)], so[b])

    issue(0, 0)

    @pl.loop(0, C, step=2)
    def _pair(j):
        for b in range(2):
            jj = j + b

            @pl.when(jj < C)
            def _process():
                wait_gather(jj, b)

                @pl.when(jj >= 1)
                def _():
                    out_desc(jj - 1, 1 - b).wait()

                @pl.when(jj + 1 < C)
                def _():
                    issue(jj + 1, 1 - b)

                @pl.loop(0, R)
                def _row(r):
                    for k in range(HW // 16):
                        sl = pl.ds(k * 16, 16)
                        sl2 = pl.ds(HW + k * 16, 16)
                        a = plsc.bitcast(bufa[b, r, sl], jnp.bfloat16)
                        c = plsc.bitcast(bufb[b, r, sl2], jnp.bfloat16)
                        obuf[b, r, sl] = plsc.bitcast(a + c, jnp.float32)

                out_desc(jj, b).start()

    out_desc(C - 1, (C - 1) % 2).wait()


def _sc_gather_add(pt, src2, dst2):
    f = pl.kernel(
        _gather_body,
        out_type=jax.ShapeDtypeStruct((N_EDGES, HW), jnp.float32),
        mesh=_mesh,
        compiler_params=pltpu.CompilerParams(needs_layout_passes=False),
        scratch_types=[
            pltpu.VMEM((C, R), jnp.int32),
            pltpu.VMEM((C, R), jnp.int32),
            pltpu.VMEM((2, R, HID), jnp.float32),
            pltpu.VMEM((2, R, HID), jnp.float32),
            pltpu.VMEM((2, R, HW), jnp.float32),
            pltpu.SemaphoreType.DMA,
            pltpu.SemaphoreType.DMA,
            pltpu.SemaphoreType.DMA,
            pltpu.SemaphoreType.DMA,
        ],
    )
    return f(pt, src2, dst2)


def _scatter_body(ef_hbm, dst_hbm, out_hbm, didx, dbuf, zbuf, si0, si1, agg_s):
    cid = lax.axis_index("c")
    sid = lax.axis_index("s")
    wid = sid * NC + cid
    si = (si0, si1)

    @pl.loop(0, R)
    def _z(r):
        for k in range(HID // 16):
            sl = pl.ds(k * 16, 16)
            zbuf[r, sl] = jnp.zeros((16,), jnp.float32)

    for t in range(STRIP // R):
        pltpu.sync_copy(zbuf, agg_s.at[pl.ds(sid * STRIP + t * R, R)])
    plsc.subcore_barrier()

    pltpu.sync_copy(dst_hbm.at[wid], didx)

    def in_desc(j, b):
        return pltpu.make_async_copy(
            ef_hbm.at[pl.ds(wid * EW + j * R, R)], dbuf.at[b], si[b])

    in_desc(0, 0).start()

    @pl.loop(0, C, step=2)
    def _pair(j):
        for b in range(2):
            jj = j + b

            @pl.when(jj < C)
            def _process():
                in_desc(jj, b).wait()

                @pl.when(jj + 1 < C)
                def _():
                    in_desc(jj + 1, 1 - b).start()

                pltpu.sync_copy(dbuf.at[b], agg_s.at[didx.at[jj]], add=True)

    plsc.subcore_barrier()
    for t in range(STRIP // R):
        rows = pl.ds(sid * STRIP + t * R, R)
        pltpu.sync_copy(agg_s.at[rows], out_hbm.at[cid].at[rows])


def _sc_scatter_add(ef, dst2):
    f = pl.kernel(
        _scatter_body,
        out_type=jax.ShapeDtypeStruct((NC, N_PAD, HID), jnp.float32),
        mesh=_mesh,
        scratch_types=[
            pltpu.VMEM((C, R), jnp.int32),
            pltpu.VMEM((2, R, HID), jnp.float32),
            pltpu.VMEM((R, HID), jnp.float32),
            pltpu.SemaphoreType.DMA,
            pltpu.SemaphoreType.DMA,
            pltpu.VMEM_SHARED((N_PAD, HID), jnp.float32),
        ],
    )
    return f(ef, dst2)


def _pack_pairs(p_u):
    lo, hi = p_u[:, :HW], p_u[:, HW:]

    def rnd(x):
        u = jax.lax.bitcast_convert_type(x, jnp.uint32)
        return (u + jnp.uint32(0x7FFF) + ((u >> 16) & jnp.uint32(1))) >> 16

    w = rnd(lo) | (rnd(hi) << 16)
    return jax.lax.bitcast_convert_type(w, jnp.float32)


def _unpack_pairs(pk):
    w = jax.lax.bitcast_convert_type(pk, jnp.uint32)
    lo = jax.lax.bitcast_convert_type(w << 16, jnp.float32)
    hi = jax.lax.bitcast_convert_type(w & jnp.uint32(0xFFFF0000), jnp.float32)
    return jnp.concatenate([lo, hi], axis=1)


def _ln(x, g, b):
    mu = jnp.mean(x, axis=-1, keepdims=True)
    var = jnp.mean((x - mu) ** 2, axis=-1, keepdims=True)
    return (x - mu) * lax.rsqrt(var + 1e-5) * g + b


def _dot(a, w):
    return jnp.dot(a, w, preferred_element_type=jnp.float32,
                   precision=lax.Precision.HIGHEST)


def _edge_block_body(h1p, ef, w1e, w2, w3, b1, b2, b3, g, beta, out):
    h1 = jax.nn.relu(_unpack_pairs(h1p[...])
                     + _dot(ef[...], w1e[...]) + b1[...])
    h2 = jax.nn.relu(_dot(h1, w2[...]) + b2[...])
    h3 = _dot(h2, w3[...]) + b3[...]
    out[...] = _ln(h3, g[...], beta[...]) + ef[...]


def _node_block_body(a0, a1, nf, w1a, w1x, w2, w3, b1, b2, b3, g, beta,
                     ws, wd, nf_out, pt_out):
    agg = a0[...] + a1[...]
    h1 = jax.nn.relu(_dot(agg, w1a[...]) + _dot(nf[...], w1x[...]) + b1[...])
    h2 = jax.nn.relu(_dot(h1, w2[...]) + b2[...])
    h3 = _dot(h2, w3[...]) + b3[...]
    y = _ln(h3, g[...], beta[...]) + nf[...]
    nf_out[...] = y
    pt_out[...] = jnp.concatenate(
        [_pack_pairs(_dot(y, ws[...])), _pack_pairs(_dot(y, wd[...]))],
        axis=1)


def _node_enc_body(x, w1, w2, w3, b1, b2, b3, g, beta, ws, wd,
                   nf_out, pt_out):
    h1 = jax.nn.relu(_dot(x[...], w1[...]) + b1[...])
    h2 = jax.nn.relu(_dot(h1, w2[...]) + b2[...])
    h3 = _dot(h2, w3[...]) + b3[...]
    y = _ln(h3, g[...], beta[...])
    nf_out[...] = y
    pt_out[...] = jnp.concatenate(
        [_pack_pairs(_dot(y, ws[...])), _pack_pairs(_dot(y, wd[...]))],
        axis=1)


def _edge_enc_body(x, w1, w2, w3, b1, b2, b3, g, beta, out):
    xb = x[...]
    acc = jnp.broadcast_to(b1[...], (xb.shape[0], HID))
    for k in range(4):
        acc = acc + xb[:, k:k + 1] * w1[k:k + 1, :]
    h1 = jax.nn.relu(acc)
    h2 = jax.nn.relu(_dot(h1, w2[...]) + b2[...])
    h3 = _dot(h2, w3[...]) + b3[...]
    out[...] = _ln(h3, g[...], beta[...])


def _decoder_body(x, w1, w2, w3, b1, b2, b3, out):
    h1 = jax.nn.relu(_dot(x[...], w1[...]) + b1[...])
    h2 = jax.nn.relu(_dot(h1, w2[...]) + b2[...])
    out[...] = _dot(h2, w3[...]) + b3[...]


def _row_spec(b, ncols):
    return pl.BlockSpec((b, ncols), lambda i: (i, 0))


def _full_spec(shape):
    return pl.BlockSpec(shape, lambda i: tuple(0 for _ in shape))


def _call_rows(body, nrows, block, in_arrays, row_args, n_out,
               out_cols=None):
    specs = []
    for a, is_row in zip(in_arrays, row_args):
        specs.append(_row_spec(block, a.shape[-1]) if is_row
                     else _full_spec(a.shape))
    if out_cols is None:
        out_cols = [HID] * n_out
    outs = [jax.ShapeDtypeStruct((nrows, c), jnp.float32) for c in out_cols]
    out_specs = [_row_spec(block, c) for c in out_cols]
    f = pl.pallas_call(
        body,
        grid=(nrows // block,),
        in_specs=specs,
        out_specs=out_specs if n_out > 1 else out_specs[0],
        out_shape=outs if n_out > 1 else outs[0],
    )
    return f(*in_arrays)


def _unpack(p):
    (w1, b1), (w2, b2), (w3, b3) = p["layers"]
    ln = p["ln"]
    out = [w1, w2, w3, b1.reshape(1, -1), b2.reshape(1, -1), b3.reshape(1, -1)]
    if ln is not None:
        g, beta = ln
        out += [g.reshape(1, -1), beta.reshape(1, -1)]
    return out


@jax.jit
def kernel(node_features, edge_features, edge_index, params):
    src2 = edge_index[0].reshape(NW, C, R)
    dst2 = edge_index[1].reshape(NW, C, R)

    eb, nb = params["edge_blocks"], params["node_blocks"]
    uperm = np.concatenate([np.arange(0, HID, 2), np.arange(1, HID, 2)])
    w1s = [p["layers"][0][0][0:HID][:, uperm] for p in eb]
    w1d = [p["layers"][0][0][HID:2 * HID][:, uperm] for p in eb]
    w1e = [p["layers"][0][0][2 * HID:][:, uperm] for p in eb]
    w2e = [p["layers"][1][0][uperm, :] for p in eb]
    b1e = [p["layers"][0][1][uperm].reshape(1, -1) for p in eb]
    w1a = [p["layers"][0][0][0:HID] for p in nb]
    w1x = [p["layers"][0][0][HID:] for p in nb]
    zero_w = jnp.zeros((HID, HID), jnp.float32)

    ne = _unpack(params["node_enc"])
    nf, pt = _call_rows(
        _node_enc_body, N_NODES, NB,
        [node_features] + ne + [w1s[0], w1d[0]],
        [True] + [False] * 10, 2,
        out_cols=[HID, HID])
    ee = _unpack(params["edge_enc"])
    ef = _call_rows(_edge_enc_body, N_EDGES, EB,
                    [edge_features] + ee, [True] + [False] * 8, 1)

    for i in range(PROC_SIZE):
        h1p = _sc_gather_add(pt, src2, dst2)
        ep = _unpack(eb[i])
        ef = _call_rows(
            _edge_block_body, N_EDGES, EB,
            [h1p, ef, w1e[i], w2e[i], ep[2], b1e[i],
             ep[4], ep[5], ep[6], ep[7]],
            [True, True] + [False] * 8, 1)
        aggp = _sc_scatter_add(ef, dst2)
        np_ = _unpack(nb[i])
        wsn = w1s[i + 1] if i + 1 < PROC_SIZE else zero_w
        wdn = w1d[i + 1] if i + 1 < PROC_SIZE else zero_w
        nf, pt = _call_rows(
            _node_block_body, N_NODES, NB,
            [aggp[0], aggp[1], nf, w1a[i], w1x[i]] + np_[1:] + [wsn, wdn],
            [True, True, True] + [False] * 11, 2,
            out_cols=[HID, HID])

    dec = _unpack(params["node_dec"])
    w3p = jnp.zeros((HID, HID), jnp.float32).at[:, :3].set(dec[2])
    b3p = jnp.zeros((1, HID), jnp.float32).at[:, :3].set(dec[5])
    out = _call_rows(_decoder_body, N_NODES, NB,
                     [nf, dec[0], dec[1], w3p, dec[3], dec[4], b3p],
                     [True] + [False] * 6, 1)
    return out[:, :3]


# speedup vs baseline: 3.1900x; 1.4467x over previous
import functools

import jax
import jax.numpy as jnp
import numpy as np
from jax import lax
from jax.experimental import pallas as pl
from jax.experimental.pallas import tpu as pltpu
from jax.experimental.pallas import tpu_sc as plsc

N_NODES = 10000
N_EDGES = 320000
HID = 128
PROC_SIZE = 15

NC, NS = 2, 16
NW = NC * NS
EH = N_EDGES // 2
EW = EH // NW
R = 40
C = EW // R
N_PAD = 10240
STRIP = N_PAD // NS
HW = HID // 2

EB = 2000
NB = 1000

_mesh = plsc.VectorSubcoreMesh(core_axis_name="c", subcore_axis_name="s")


def _gather_body(pt_hbm, src_hbm, dst_hbm, out_hbm,
                 sidx, didx, bufa, bufb, obuf, sg0, sg1, so0, so1):
    cid = lax.axis_index("c")
    sid = lax.axis_index("s")
    wid = sid * NC + cid
    pltpu.sync_copy(src_hbm.at[wid], sidx)
    pltpu.sync_copy(dst_hbm.at[wid], didx)
    base = wid * EW
    sg = (sg0, sg1)
    so = (so0, so1)

    def issue(j, b):
        pltpu.async_copy(pt_hbm.at[sidx.at[j]], bufa.at[b], sg[b])
        pltpu.async_copy(pt_hbm.at[didx.at[j]], bufb.at[b], sg[b])

    def wait_gather(j, b):
        pltpu.make_async_copy(pt_hbm.at[sidx.at[j]], bufa.at[b], sg[b]).wait()
        pltpu.make_async_copy(pt_hbm.at[didx.at[j]], bufb.at[b], sg[b]).wait()

    def out_desc(j, b):
        return pltpu.make_async_copy(
            obuf.at[b], out_hbm.at[pl.ds(base + j * R, R)], so[b])

    issue(0, 0)

    @pl.loop(0, C, step=2)
    def _pair(j):
        for b in range(2):
            jj = j + b

            @pl.when(jj < C)
            def _process():
                wait_gather(jj, b)

                @pl.when(jj >= 1)
                def _():
                    out_desc(jj - 1, 1 - b).wait()

                @pl.when(jj + 1 < C)
                def _():
                    issue(jj + 1, 1 - b)

                @pl.loop(0, R, unroll=8)
                def _row(r):
                    for k in range(HW // 16):
                        sl = pl.ds(k * 16, 16)
                        sl2 = pl.ds(HW + k * 16, 16)
                        a = plsc.bitcast(bufa[b, r, sl], jnp.bfloat16)
                        c = plsc.bitcast(bufb[b, r, sl2], jnp.bfloat16)
                        obuf[b, r, sl] = plsc.bitcast(a + c, jnp.float32)

                out_desc(jj, b).start()

    out_desc(C - 1, (C - 1) % 2).wait()


def _sc_gather_add(pt, src2, dst2):
    f = pl.kernel(
        _gather_body,
        out_type=jax.ShapeDtypeStruct((EH, HW), jnp.float32),
        mesh=_mesh,
        compiler_params=pltpu.CompilerParams(needs_layout_passes=False),
        scratch_types=[
            pltpu.VMEM((C, R), jnp.int32),
            pltpu.VMEM((C, R), jnp.int32),
            pltpu.VMEM((2, R, HID), jnp.float32),
            pltpu.VMEM((2, R, HID), jnp.float32),
            pltpu.VMEM((2, R, HW), jnp.float32),
            pltpu.SemaphoreType.DMA,
            pltpu.SemaphoreType.DMA,
            pltpu.SemaphoreType.DMA,
            pltpu.SemaphoreType.DMA,
        ],
    )
    return f(pt, src2, dst2)


def _scatter_body(ef_hbm, dst_hbm, out_hbm, didx, dbuf, zbuf,
                  si0, si1, sa0, sa1, agg_s):
    cid = lax.axis_index("c")
    sid = lax.axis_index("s")
    wid = sid * NC + cid
    si = (si0, si1)
    sa = (sa0, sa1)

    @pl.loop(0, R)
    def _z(r):
        for k in range(HID // 16):
            sl = pl.ds(k * 16, 16)
            zbuf[r, sl] = jnp.zeros((16,), jnp.float32)

    for t in range(STRIP // R):
        pltpu.sync_copy(zbuf, agg_s.at[pl.ds(sid * STRIP + t * R, R)])
    plsc.subcore_barrier()

    pltpu.sync_copy(dst_hbm.at[wid], didx)

    def in_desc(j, b):
        return pltpu.make_async_copy(
            ef_hbm.at[pl.ds(wid * EW + j * R, R)], dbuf.at[b], si[b])

    def add_wait(j, b):
        pltpu.make_async_copy(dbuf.at[b], agg_s.at[didx.at[j]], sa[b]).wait()

    in_desc(0, 0).start()

    @pl.loop(0, C, step=2)
    def _pair(j):
        for b in range(2):
            jj = j + b

            @pl.when(jj < C)
            def _process():
                in_desc(jj, b).wait()

                @pl.when(jj >= 1)
                def _():
                    add_wait(jj - 1, 1 - b)

                @pl.when(jj + 1 < C)
                def _():
                    in_desc(jj + 1, 1 - b).start()

                pltpu.async_copy(dbuf.at[b], agg_s.at[didx.at[jj]], sa[b],
                                 add=True)

    add_wait(C - 1, (C - 1) % 2)
    plsc.subcore_barrier()
    for t in range(STRIP // R):
        rows = pl.ds(sid * STRIP + t * R, R)
        pltpu.sync_copy(agg_s.at[rows], out_hbm.at[cid].at[rows])


def _sc_scatter_add(ef, dst2):
    f = pl.kernel(
        _scatter_body,
        out_type=jax.ShapeDtypeStruct((NC, N_PAD, HID), jnp.float32),
        mesh=_mesh,
        scratch_types=[
            pltpu.VMEM((C, R), jnp.int32),
            pltpu.VMEM((2, R, HID), jnp.float32),
            pltpu.VMEM((R, HID), jnp.float32),
            pltpu.SemaphoreType.DMA,
            pltpu.SemaphoreType.DMA,
            pltpu.SemaphoreType.DMA,
            pltpu.SemaphoreType.DMA,
            pltpu.VMEM_SHARED((N_PAD, HID), jnp.float32),
        ],
    )
    return f(ef, dst2)


def _pack_pairs(p_u):
    lo, hi = p_u[:, :HW], p_u[:, HW:]

    def rnd(x):
        u = jax.lax.bitcast_convert_type(x, jnp.uint32)
        return (u + jnp.uint32(0x7FFF) + ((u >> 16) & jnp.uint32(1))) >> 16

    w = rnd(lo) | (rnd(hi) << 16)
    return jax.lax.bitcast_convert_type(w, jnp.float32)


def _unpack_pairs(pk):
    w = jax.lax.bitcast_convert_type(pk, jnp.uint32)
    lo = jax.lax.bitcast_convert_type(w << 16, jnp.float32)
    hi = jax.lax.bitcast_convert_type(w & jnp.uint32(0xFFFF0000), jnp.float32)
    return jnp.concatenate([lo, hi], axis=1)


def _ln(x, g, b):
    mu = jnp.mean(x, axis=-1, keepdims=True)
    var = jnp.mean((x - mu) ** 2, axis=-1, keepdims=True)
    return (x - mu) / jnp.sqrt(var + 1e-5) * g + b


def _dot(a, w):
    ah = a.astype(jnp.bfloat16)
    al = (a - ah.astype(jnp.float32)).astype(jnp.bfloat16)
    wh = w.astype(jnp.bfloat16)
    wl = (w - wh.astype(jnp.float32)).astype(jnp.bfloat16)
    f32 = jnp.float32
    return (jnp.dot(ah, wh, preferred_element_type=f32)
            + jnp.dot(ah, wl, preferred_element_type=f32)
            + jnp.dot(al, wh, preferred_element_type=f32))


def _edge_block_body(h1p, ef, w1e, w2, w3, b1, b2, b3, g, beta, out):
    h1 = jax.nn.relu(_unpack_pairs(h1p[...])
                     + _dot(ef[...], w1e[...]) + b1[...])
    h2 = jax.nn.relu(_dot(h1, w2[...]) + b2[...])
    h3 = _dot(h2, w3[...]) + b3[...]
    out[...] = _ln(h3, g[...], beta[...]) + ef[...]


def _node_block_body(a0, a1, a2, a3, nf, w1a, w1x, w2, w3, b1, b2, b3, g,
                     beta, ws, wd, nf_out, pt_out):
    agg = (a0[...] + a1[...]) + (a2[...] + a3[...])
    h1 = jax.nn.relu(_dot(agg, w1a[...]) + _dot(nf[...], w1x[...]) + b1[...])
    h2 = jax.nn.relu(_dot(h1, w2[...]) + b2[...])
    h3 = _dot(h2, w3[...]) + b3[...]
    y = _ln(h3, g[...], beta[...]) + nf[...]
    nf_out[...] = y
    pt_out[...] = jnp.concatenate(
        [_pack_pairs(_dot(y, ws[...])), _pack_pairs(_dot(y, wd[...]))],
        axis=1)


def _node_enc_body(x, w1, w2, w3, b1, b2, b3, g, beta, ws, wd,
                   nf_out, pt_out):
    h1 = jax.nn.relu(_dot(x[...], w1[...]) + b1[...])
    h2 = jax.nn.relu(_dot(h1, w2[...]) + b2[...])
    h3 = _dot(h2, w3[...]) + b3[...]
    y = _ln(h3, g[...], beta[...])
    nf_out[...] = y
    pt_out[...] = jnp.concatenate(
        [_pack_pairs(_dot(y, ws[...])), _pack_pairs(_dot(y, wd[...]))],
        axis=1)


def _edge_enc_body(x, w1, w2, w3, b1, b2, b3, g, beta, out):
    xb = x[...]
    acc = jnp.broadcast_to(b1[...], (xb.shape[0], HID))
    for k in range(4):
        acc = acc + xb[:, k:k + 1] * w1[k:k + 1, :]
    h1 = jax.nn.relu(acc)
    h2 = jax.nn.relu(_dot(h1, w2[...]) + b2[...])
    h3 = _dot(h2, w3[...]) + b3[...]
    out[...] = _ln(h3, g[...], beta[...])


def _decoder_body(x, w1, w2, w3, b1, b2, b3, out):
    h1 = jax.nn.relu(_dot(x[...], w1[...]) + b1[...])
    h2 = jax.nn.relu(_dot(h1, w2[...]) + b2[...])
    out[...] = _dot(h2, w3[...]) + b3[...]


def _row_spec(b, ncols):
    return pl.BlockSpec((b, ncols), lambda i: (i, 0))


def _full_spec(shape):
    return pl.BlockSpec(shape, lambda i: tuple(0 for _ in shape))


def _call_rows(body, nrows, block, in_arrays, row_args, n_out,
               out_cols=None):
    specs = []
    for a, is_row in zip(in_arrays, row_args):
        specs.append(_row_spec(block, a.shape[-1]) if is_row
                     else _full_spec(a.shape))
    if out_cols is None:
        out_cols = [HID] * n_out
    outs = [jax.ShapeDtypeStruct((nrows, c), jnp.float32) for c in out_cols]
    out_specs = [_row_spec(block, c) for c in out_cols]
    f = pl.pallas_call(
        body,
        grid=(nrows // block,),
        in_specs=specs,
        out_specs=out_specs if n_out > 1 else out_specs[0],
        out_shape=outs if n_out > 1 else outs[0],
    )
    return f(*in_arrays)


def _unpack(p):
    (w1, b1), (w2, b2), (w3, b3) = p["layers"]
    ln = p["ln"]
    out = [w1, w2, w3, b1.reshape(1, -1), b2.reshape(1, -1), b3.reshape(1, -1)]
    if ln is not None:
        g, beta = ln
        out += [g.reshape(1, -1), beta.reshape(1, -1)]
    return out


@jax.jit
def kernel(node_features, edge_features, edge_index, params):
    srcA = edge_index[0][:EH].reshape(NW, C, R)
    srcB = edge_index[0][EH:].reshape(NW, C, R)
    dstA = edge_index[1][:EH].reshape(NW, C, R)
    dstB = edge_index[1][EH:].reshape(NW, C, R)

    eb, nb = params["edge_blocks"], params["node_blocks"]
    uperm = np.concatenate([np.arange(0, HID, 2), np.arange(1, HID, 2)])
    w1s = [p["layers"][0][0][0:HID][:, uperm] for p in eb]
    w1d = [p["layers"][0][0][HID:2 * HID][:, uperm] for p in eb]
    w1e = [p["layers"][0][0][2 * HID:][:, uperm] for p in eb]
    w2e = [p["layers"][1][0][uperm, :] for p in eb]
    b1e = [p["layers"][0][1][uperm].reshape(1, -1) for p in eb]
    w1a = [p["layers"][0][0][0:HID] for p in nb]
    w1x = [p["layers"][0][0][HID:] for p in nb]
    zero_w = jnp.zeros((HID, HID), jnp.float32)

    ne = _unpack(params["node_enc"])
    nf, pt = _call_rows(
        _node_enc_body, N_NODES, NB,
        [node_features] + ne + [w1s[0], w1d[0]],
        [True] + [False] * 10, 2,
        out_cols=[HID, HID])
    ee = _unpack(params["edge_enc"])
    efA = _call_rows(_edge_enc_body, EH, EB,
                     [edge_features[:EH]] + ee, [True] + [False] * 8, 1)
    efB = _call_rows(_edge_enc_body, EH, EB,
                     [edge_features[EH:]] + ee, [True] + [False] * 8, 1)

    for i in range(PROC_SIZE):
        ep = _unpack(eb[i])
        edge_w = [w1e[i], w2e[i], ep[2], b1e[i], ep[4], ep[5], ep[6], ep[7]]
        h1pA = _sc_gather_add(pt, srcA, dstA)
        h1pB = _sc_gather_add(pt, srcB, dstB)
        efA = _call_rows(_edge_block_body, EH, EB,
                         [h1pA, efA] + edge_w, [True, True] + [False] * 8, 1)
        aggA = _sc_scatter_add(efA, dstA)
        efB = _call_rows(_edge_block_body, EH, EB,
                         [h1pB, efB] + edge_w, [True, True] + [False] * 8, 1)
        aggB = _sc_scatter_add(efB, dstB)
        np_ = _unpack(nb[i])
        wsn = w1s[i + 1] if i + 1 < PROC_SIZE else zero_w
        wdn = w1d[i + 1] if i + 1 < PROC_SIZE else zero_w
        nf, pt = _call_rows(
            _node_block_body, N_NODES, NB,
            [aggA[0], aggA[1], aggB[0], aggB[1], nf, w1a[i], w1x[i]]
            + np_[1:] + [wsn, wdn],
            [True] * 5 + [False] * 11, 2,
            out_cols=[HID, HID])

    dec = _unpack(params["node_dec"])
    w3p = jnp.zeros((HID, HID), jnp.float32).at[:, :3].set(dec[2])
    b3p = jnp.zeros((1, HID), jnp.float32).at[:, :3].set(dec[5])
    out = _call_rows(_decoder_body, N_NODES, NB,
                     [nf, dec[0], dec[1], w3p, dec[3], dec[4], b3p],
                     [True] + [False] * 6, 1)
    return out[:, :3]


# speedup vs baseline: 4.1310x; 1.2950x over previous
import functools

import jax
import jax.numpy as jnp
import numpy as np
from jax import lax
from jax.experimental import pallas as pl
from jax.experimental.pallas import tpu as pltpu
from jax.experimental.pallas import tpu_sc as plsc

N_NODES = 10000
N_EDGES = 320000
HID = 128
PROC_SIZE = 15

NC, NS = 2, 16
NW = NC * NS
EH = N_EDGES // 2
EW = EH // NW
R = 40
C = EW // R
D = 4
S = C // D
D2 = 2
S2 = (C // D2)
P = 2 * R
N_PAD = 10240
STRIP = N_PAD // NS
HW = HID // 2

EB = 2000
NB = 1000

_mesh = plsc.VectorSubcoreMesh(core_axis_name="c", subcore_axis_name="s")


def _gather_body(ps_hbm, pd_hbm, src_hbm, dst_hbm, out_hbm,
                 sidx, didx, bufa, bufb, obuf, sg0, sg1, so0):
    cid = lax.axis_index("c")
    sid = lax.axis_index("s")
    wid = sid * NC + cid
    base = wid * EW
    sg = (sg0, sg1)

    pltpu.sync_copy(src_hbm.at[wid], sidx)
    pltpu.sync_copy(dst_hbm.at[wid], didx)

    def issue(s, b):
        for d in range(D):
            rows = pl.ds(d * R, R)
            pltpu.async_copy(ps_hbm.at[sidx.at[s * D + d]],
                             bufa.at[b, rows], sg[b])
            pltpu.async_copy(pd_hbm.at[didx.at[s * D + d]],
                             bufb.at[b, rows], sg[b])

    def wait_gather(s, b):
        for d in range(D):
            rows = pl.ds(d * R, R)
            pltpu.make_async_copy(
                ps_hbm.at[sidx.at[s * D + d]], bufa.at[b, rows],
                sg[b]).wait()
            pltpu.make_async_copy(
                pd_hbm.at[didx.at[s * D + d]], bufb.at[b, rows],
                sg[b]).wait()

    def out_desc(s, part):
        return pltpu.make_async_copy(
            obuf, out_hbm.at[pl.ds(base + s * D * R + part * P, P)], so0)

    def add_rows(b, lo):
        @pl.loop(0, P, unroll=8)
        def _row(r):
            for k in range(HID // 16):
                sl = pl.ds(k * 16, 16)
                obuf[r, sl] = bufa[b, lo + r, sl] + bufb[b, lo + r, sl]

    issue(0, 0)

    @pl.loop(0, S, step=2)
    def _pair(s):
        for b in range(2):
            ss = s + b

            @pl.when(ss < S)
            def _process():
                wait_gather(ss, b)

                @pl.when(ss >= 1)
                def _():
                    out_desc(ss - 1, 1).wait()

                @pl.when(ss + 1 < S)
                def _():
                    issue(ss + 1, 1 - b)

                add_rows(b, 0)
                out_desc(ss, 0).start()
                out_desc(ss, 0).wait()
                add_rows(b, P)
                out_desc(ss, 1).start()

    out_desc(S - 1, 1).wait()
    jlast = C - 1
    r0 = pl.ds(0, R)
    pltpu.async_copy(ps_hbm.at[sidx.at[jlast]], bufa.at[0, r0], sg0)
    pltpu.make_async_copy(ps_hbm.at[sidx.at[jlast]], bufa.at[0, r0],
                          sg0).wait()
    pltpu.async_copy(pd_hbm.at[didx.at[jlast]], bufb.at[0, r0], sg0)
    pltpu.make_async_copy(pd_hbm.at[didx.at[jlast]], bufb.at[0, r0],
                          sg0).wait()

    @pl.loop(0, R, unroll=8)
    def _rowl(r):
        for k in range(HID // 16):
            sl = pl.ds(k * 16, 16)
            obuf[r, sl] = bufa[0, r, sl] + bufb[0, r, sl]

    pltpu.sync_copy(obuf.at[r0], out_hbm.at[pl.ds(base + jlast * R, R)])


def _sc_gather_add(ps, pd, src2, dst2):
    f = pl.kernel(
        _gather_body,
        out_type=jax.ShapeDtypeStruct((EH, HID), jnp.float32),
        mesh=_mesh,
        scratch_types=[
            pltpu.VMEM((C, R), jnp.int32),
            pltpu.VMEM((C, R), jnp.int32),
            pltpu.VMEM((2, D * R, HID), jnp.float32),
            pltpu.VMEM((2, D * R, HID), jnp.float32),
            pltpu.VMEM((P, HID), jnp.float32),
            pltpu.SemaphoreType.DMA,
            pltpu.SemaphoreType.DMA,
            pltpu.SemaphoreType.DMA,
        ],
    )
    return f(ps, pd, src2, dst2)


def _scatter_body(ef_hbm, dst_hbm, out_hbm, didx, dbuf, zbuf,
                  si0, si1, sa0, sa1, agg_s):
    cid = lax.axis_index("c")
    sid = lax.axis_index("s")
    wid = sid * NC + cid
    si = (si0, si1)
    sa = (sa0, sa1)

    @pl.loop(0, R)
    def _z(r):
        for k in range(HID // 16):
            sl = pl.ds(k * 16, 16)
            zbuf[r, sl] = jnp.zeros((16,), jnp.float32)

    for t in range(STRIP // R):
        pltpu.sync_copy(zbuf, agg_s.at[pl.ds(sid * STRIP + t * R, R)])
    plsc.subcore_barrier()

    pltpu.sync_copy(dst_hbm.at[wid], didx)

    def in_desc(s, b):
        return pltpu.make_async_copy(
            ef_hbm.at[pl.ds(wid * EW + s * D2 * R, D2 * R)], dbuf.at[b],
            si[b])

    def add_wait(s, b):
        for d in range(D2):
            rows = pl.ds(d * R, R)
            pltpu.make_async_copy(
                dbuf.at[b, rows], agg_s.at[didx.at[s * D2 + d]],
                sa[b]).wait()

    in_desc(0, 0).start()

    @pl.loop(0, S2, step=2)
    def _pair(s):
        for b in range(2):
            ss = s + b

            @pl.when(ss < S2)
            def _process():
                in_desc(ss, b).wait()

                @pl.when(ss >= 1)
                def _():
                    add_wait(ss - 1, 1 - b)

                @pl.when(ss + 1 < S2)
                def _():
                    in_desc(ss + 1, 1 - b).start()

                for d in range(D2):
                    rows = pl.ds(d * R, R)
                    pltpu.async_copy(dbuf.at[b, rows],
                                     agg_s.at[didx.at[ss * D2 + d]],
                                     sa[b], add=True)

    add_wait(S2 - 1, (S2 - 1) % 2)
    rows0 = pl.ds(0, R)
    pltpu.sync_copy(ef_hbm.at[pl.ds(wid * EW + (C - 1) * R, R)],
                    dbuf.at[0, rows0])
    pltpu.sync_copy(dbuf.at[0, rows0], agg_s.at[didx.at[C - 1]], add=True)
    plsc.subcore_barrier()
    for t in range(STRIP // R):
        rows = pl.ds(sid * STRIP + t * R, R)
        pltpu.sync_copy(agg_s.at[rows], out_hbm.at[cid].at[rows])


def _sc_scatter_add(ef, dst2):
    f = pl.kernel(
        _scatter_body,
        out_type=jax.ShapeDtypeStruct((NC, N_PAD, HID), jnp.float32),
        mesh=_mesh,
        scratch_types=[
            pltpu.VMEM((C, R), jnp.int32),
            pltpu.VMEM((2, D2 * R, HID), jnp.float32),
            pltpu.VMEM((R, HID), jnp.float32),
            pltpu.SemaphoreType.DMA,
            pltpu.SemaphoreType.DMA,
            pltpu.SemaphoreType.DMA,
            pltpu.SemaphoreType.DMA,
            pltpu.VMEM_SHARED((N_PAD, HID), jnp.float32),
        ],
    )
    return f(ef, dst2)


def _ln(x, g, b):
    mu = jnp.mean(x, axis=-1, keepdims=True)
    var = jnp.mean((x - mu) ** 2, axis=-1, keepdims=True)
    return (x - mu) / jnp.sqrt(var + 1e-5) * g + b


def _dot(a, w):
    return jnp.dot(a.astype(jnp.bfloat16), w.astype(jnp.bfloat16),
                   preferred_element_type=jnp.float32)


def _edge_block_body(h1p, ef, w1e, w2, w3, b1, b2, b3, g, beta, out):
    h1 = jax.nn.relu(h1p[...] + _dot(ef[...], w1e[...]) + b1[...])
    h2 = jax.nn.relu(_dot(h1, w2[...]) + b2[...])
    h3 = _dot(h2, w3[...]) + b3[...]
    out[...] = _ln(h3, g[...], beta[...]) + ef[...]


def _node_block_body(a0, a1, a2, a3, nf, w1a, w1x, w2, w3, b1, b2, b3, g,
                     beta, ws, wd, nf_out, ps_out, pd_out):
    agg = (a0[...] + a1[...]) + (a2[...] + a3[...])
    h1 = jax.nn.relu(_dot(agg, w1a[...]) + _dot(nf[...], w1x[...]) + b1[...])
    h2 = jax.nn.relu(_dot(h1, w2[...]) + b2[...])
    h3 = _dot(h2, w3[...]) + b3[...]
    y = _ln(h3, g[...], beta[...]) + nf[...]
    nf_out[...] = y
    ps_out[...] = _dot(y, ws[...])
    pd_out[...] = _dot(y, wd[...])


def _node_enc_body(x, w1, w2, w3, b1, b2, b3, g, beta, ws, wd,
                   nf_out, ps_out, pd_out):
    h1 = jax.nn.relu(_dot(x[...], w1[...]) + b1[...])
    h2 = jax.nn.relu(_dot(h1, w2[...]) + b2[...])
    h3 = _dot(h2, w3[...]) + b3[...]
    y = _ln(h3, g[...], beta[...])
    nf_out[...] = y
    ps_out[...] = _dot(y, ws[...])
    pd_out[...] = _dot(y, wd[...])


def _edge_enc_body(x, w1, w2, w3, b1, b2, b3, g, beta, out):
    xb = x[...]
    acc = jnp.broadcast_to(b1[...], (xb.shape[0], HID))
    for k in range(4):
        acc = acc + xb[:, k:k + 1] * w1[k:k + 1, :]
    h1 = jax.nn.relu(acc)
    h2 = jax.nn.relu(_dot(h1, w2[...]) + b2[...])
    h3 = _dot(h2, w3[...]) + b3[...]
    out[...] = _ln(h3, g[...], beta[...])


def _decoder_body(x, w1, w2, w3, b1, b2, b3, out):
    h1 = jax.nn.relu(_dot(x[...], w1[...]) + b1[...])
    h2 = jax.nn.relu(_dot(h1, w2[...]) + b2[...])
    out[...] = _dot(h2, w3[...]) + b3[...]


def _row_spec(b, ncols):
    return pl.BlockSpec((b, ncols), lambda i: (i, 0))


def _full_spec(shape):
    return pl.BlockSpec(shape, lambda i: tuple(0 for _ in shape))


def _call_rows(body, nrows, block, in_arrays, row_args, n_out,
               out_cols=None):
    specs = []
    for a, is_row in zip(in_arrays, row_args):
        specs.append(_row_spec(block, a.shape[-1]) if is_row
                     else _full_spec(a.shape))
    if out_cols is None:
        out_cols = [HID] * n_out
    outs = [jax.ShapeDtypeStruct((nrows, c), jnp.float32) for c in out_cols]
    out_specs = [_row_spec(block, c) for c in out_cols]
    f = pl.pallas_call(
        body,
        grid=(nrows // block,),
        in_specs=specs,
        out_specs=out_specs if n_out > 1 else out_specs[0],
        out_shape=outs if n_out > 1 else outs[0],
    )
    return f(*in_arrays)


def _unpack(p):
    (w1, b1), (w2, b2), (w3, b3) = p["layers"]
    ln = p["ln"]
    out = [w1, w2, w3, b1.reshape(1, -1), b2.reshape(1, -1), b3.reshape(1, -1)]
    if ln is not None:
        g, beta = ln
        out += [g.reshape(1, -1), beta.reshape(1, -1)]
    return out


@jax.jit
def kernel(node_features, edge_features, edge_index, params):
    srcA = edge_index[0][:EH].reshape(NW, C, R)
    srcB = edge_index[0][EH:].reshape(NW, C, R)
    dstA = edge_index[1][:EH].reshape(NW, C, R)
    dstB = edge_index[1][EH:].reshape(NW, C, R)

    eb, nb = params["edge_blocks"], params["node_blocks"]
    w1s = [p["layers"][0][0][0:HID] for p in eb]
    w1d = [p["layers"][0][0][HID:2 * HID] for p in eb]
    w1e = [p["layers"][0][0][2 * HID:] for p in eb]
    w1a = [p["layers"][0][0][0:HID] for p in nb]
    w1x = [p["layers"][0][0][HID:] for p in nb]
    zero_w = jnp.zeros((HID, HID), jnp.float32)

    ne = _unpack(params["node_enc"])
    nf, ps, pd = _call_rows(
        _node_enc_body, N_NODES, NB,
        [node_features] + ne + [w1s[0], w1d[0]],
        [True] + [False] * 10, 3,
        out_cols=[HID, HID, HID])
    ee = _unpack(params["edge_enc"])
    efA = _call_rows(_edge_enc_body, EH, EB,
                     [edge_features[:EH]] + ee, [True] + [False] * 8, 1)
    efB = _call_rows(_edge_enc_body, EH, EB,
                     [edge_features[EH:]] + ee, [True] + [False] * 8, 1)

    for i in range(PROC_SIZE):
        ep = _unpack(eb[i])
        edge_w = [w1e[i]] + ep[1:]
        h1pA = _sc_gather_add(ps, pd, srcA, dstA)
        h1pB = _sc_gather_add(ps, pd, srcB, dstB)
        efA = _call_rows(_edge_block_body, EH, EB,
                         [h1pA, efA] + edge_w, [True, True] + [False] * 8, 1)
        aggA = _sc_scatter_add(efA, dstA)
        efB = _call_rows(_edge_block_body, EH, EB,
                         [h1pB, efB] + edge_w, [True, True] + [False] * 8, 1)
        aggB = _sc_scatter_add(efB, dstB)
        np_ = _unpack(nb[i])
        wsn = w1s[i + 1] if i + 1 < PROC_SIZE else zero_w
        wdn = w1d[i + 1] if i + 1 < PROC_SIZE else zero_w
        nf, ps, pd = _call_rows(
            _node_block_body, N_NODES, NB,
            [aggA[0], aggA[1], aggB[0], aggB[1], nf, w1a[i], w1x[i]]
            + np_[1:] + [wsn, wdn],
            [True] * 5 + [False] * 11, 3,
            out_cols=[HID, HID, HID])

    dec = _unpack(params["node_dec"])
    w3p = jnp.zeros((HID, HID), jnp.float32).at[:, :3].set(dec[2])
    b3p = jnp.zeros((1, HID), jnp.float32).at[:, :3].set(dec[5])
    out = _call_rows(_decoder_body, N_NODES, NB,
                     [nf, dec[0], dec[1], w3p, dec[3], dec[4], b3p],
                     [True] + [False] * 6, 1)
    return out[:, :3]
